# Initial kernel scaffold; baseline (speedup 1.0000x reference)
import jax
import jax.numpy as jnp
from jax.experimental import pallas as pl


def kernel(x, emb, W, b):
    raise NotImplementedError("write your pallas kernel here")


# speedup vs baseline: 2.1913x; 2.1913x over previous
import functools

import jax
import jax.numpy as jnp
from jax import lax
from jax.experimental import pallas as pl
from jax.experimental.pallas import tpu as pltpu
from jax.experimental.pallas import tpu_sc as plsc

VOCAB = 1000000
D = 64
B = 16384
H = 200

_NC = 2
_NS = 16
_NW = _NC * _NS
_SPW = B // _NW
_S = 2
_CHUNK = 100
_NCH = _S * H // _CHUNK
_GROUPS = _SPW // _S


def _pool_body(x2_hbm, emb_hbm, out_hbm, idx_v, rows_v, out_v, sem):
    wid = lax.axis_index("s") * _NC + lax.axis_index("c")
    base = wid * _SPW

    def group(g, carry):
        row0 = (base + g * _S) * (H // _CHUNK)
        pltpu.sync_copy(x2_hbm.at[pl.ds(row0, _NCH)], idx_v)
        cps = [
            pltpu.async_copy(
                emb_hbm.at[idx_v.at[j]],
                rows_v.at[pl.ds(j * _CHUNK, _CHUNK)],
                sem,
            )
            for j in range(_NCH)
        ]
        for c in cps:
            c.wait()
        for s in range(_S):
            r0 = s * H

            def red(r, acc):
                return tuple(
                    acc[k] + rows_v[r0 + r, pl.ds(16 * k, 16)] for k in range(4)
                )

            acc = lax.fori_loop(
                0, H, red,
                tuple(jnp.zeros((16,), jnp.float32) for _ in range(4)),
                unroll=8,
            )
            for k in range(4):
                out_v[g * _S + s, pl.ds(16 * k, 16)] = acc[k]
        return carry

    lax.fori_loop(0, _GROUPS, group, 0)
    pltpu.sync_copy(out_v, out_hbm.at[pl.ds(base, _SPW)])


def _pool(x, emb):
    x2 = x.reshape(B * H // _CHUNK, _CHUNK)
    mesh = plsc.VectorSubcoreMesh(core_axis_name="c", subcore_axis_name="s")
    fn = functools.partial(
        pl.kernel,
        mesh=mesh,
        compiler_params=pltpu.CompilerParams(use_tc_tiling_on_sc=False),
        out_type=jax.ShapeDtypeStruct((B, D), jnp.float32),
        scratch_types=[
            pltpu.VMEM((_NCH, _CHUNK), jnp.int32),
            pltpu.VMEM((_S * H, D), jnp.float32),
            pltpu.VMEM((_SPW, D), jnp.float32),
            pltpu.SemaphoreType.DMA,
        ],
    )(_pool_body)
    return fn(x2, emb)


def _dense_body(h_ref, w_ref, b_ref, o_ref):
    o_ref[...] = jnp.tanh(
        jnp.dot(h_ref[...], w_ref[...], preferred_element_type=jnp.float32)
        + b_ref[...]
    )


def _dense(h, W, b):
    blk = 2048
    return pl.pallas_call(
        _dense_body,
        grid=(B // blk,),
        in_specs=[
            pl.BlockSpec((blk, D), lambda i: (i, 0)),
            pl.BlockSpec((D, D), lambda i: (0, 0)),
            pl.BlockSpec((1, D), lambda i: (0, 0)),
        ],
        out_specs=pl.BlockSpec((blk, D), lambda i: (i, 0)),
        out_shape=jax.ShapeDtypeStruct((B, D), jnp.float32),
    )(h, W, b.reshape(1, D))


def kernel(x, emb, W, b):
    pooled = _pool(x, emb)
    return _dense(pooled, W, b)


# speedup vs baseline: 3.1585x; 1.4414x over previous
import functools

import jax
import jax.numpy as jnp
from jax import lax
from jax.experimental import pallas as pl
from jax.experimental.pallas import tpu as pltpu
from jax.experimental.pallas import tpu_sc as plsc

VOCAB = 1000000
D = 64
B = 16384
H = 200

_NC = 2
_NS = 16
_NW = _NC * _NS
_SPW = B // _NW
_S = 2
_CHUNK = 100
_NCH = _S * H // _CHUNK
_GROUPS = _SPW // _S


def _pool_body(x2_hbm, emb_hbm, out_hbm, idx_v, rows_v, out_v,
               sem_r0, sem_r1, sem_i0, sem_i1):
    wid = lax.axis_index("s") * _NC + lax.axis_index("c")
    base = wid * _SPW
    sems_r = (sem_r0, sem_r1)
    sems_i = (sem_i0, sem_i1)

    def idx_row0(g):
        return (base + g * _S) * (H // _CHUNK)

    def idx_fetch(g, b):
        pltpu.async_copy(
            x2_hbm.at[pl.ds(idx_row0(g), _NCH)], idx_v.at[b], sems_i[b]
        )

    def wait_idx(b):
        pltpu.make_async_copy(
            x2_hbm.at[pl.ds(0, _NCH)], idx_v.at[b], sems_i[b]
        ).wait()

    def fire(b):
        for j in range(_NCH):
            pltpu.async_copy(
                emb_hbm.at[idx_v.at[b, j]],
                rows_v.at[b, pl.ds(j * _CHUNK, _CHUNK)],
                sems_r[b],
            )

    def wait_rows(b):
        pltpu.make_async_copy(
            emb_hbm.at[pl.ds(0, _S * H)], rows_v.at[b], sems_r[b]
        ).wait()

    def reduce(g, b):
        for s in range(_S):
            r0 = s * H

            def red(r, acc):
                return tuple(
                    acc[k] + rows_v[b, r0 + r, pl.ds(16 * k, 16)]
                    for k in range(4)
                )

            acc = lax.fori_loop(
                0, H, red,
                tuple(jnp.zeros((16,), jnp.float32) for _ in range(4)),
                unroll=8,
            )
            for k in range(4):
                out_v[g * _S + s, pl.ds(16 * k, 16)] = acc[k]

    pltpu.sync_copy(x2_hbm.at[pl.ds(idx_row0(0), _NCH)], idx_v.at[0])
    fire(0)
    idx_fetch(1, 1)

    def pair(gg, carry):
        for b in range(2):
            g = 2 * gg + b
            nb = 1 - b

            @pl.when(g + 1 < _GROUPS)
            def _():
                wait_idx(nb)
                fire(nb)

            wait_rows(b)

            @pl.when(g + 2 < _GROUPS)
            def _():
                idx_fetch(g + 2, b)

            reduce(g, b)
        return carry

    lax.fori_loop(0, _GROUPS // 2, pair, 0)
    pltpu.sync_copy(out_v, out_hbm.at[pl.ds(base, _SPW)])


def _pool(x, emb):
    x2 = x.reshape(B * H // _CHUNK, _CHUNK)
    mesh = plsc.VectorSubcoreMesh(core_axis_name="c", subcore_axis_name="s")
    fn = functools.partial(
        pl.kernel,
        mesh=mesh,
        compiler_params=pltpu.CompilerParams(use_tc_tiling_on_sc=False),
        out_type=jax.ShapeDtypeStruct((B, D), jnp.float32),
        scratch_types=[
            pltpu.VMEM((2, _NCH, _CHUNK), jnp.int32),
            pltpu.VMEM((2, _S * H, D), jnp.float32),
            pltpu.VMEM((_SPW, D), jnp.float32),
            pltpu.SemaphoreType.DMA,
            pltpu.SemaphoreType.DMA,
            pltpu.SemaphoreType.DMA,
            pltpu.SemaphoreType.DMA,
        ],
    )(_pool_body)
    return fn(x2, emb)


def _dense_body(h_ref, w_ref, b_ref, o_ref):
    o_ref[...] = jnp.tanh(
        jnp.dot(h_ref[...], w_ref[...], preferred_element_type=jnp.float32)
        + b_ref[...]
    )


def _dense(h, W, b):
    blk = 2048
    return pl.pallas_call(
        _dense_body,
        grid=(B // blk,),
        in_specs=[
            pl.BlockSpec((blk, D), lambda i: (i, 0)),
            pl.BlockSpec((D, D), lambda i: (0, 0)),
            pl.BlockSpec((1, D), lambda i: (0, 0)),
        ],
        out_specs=pl.BlockSpec((blk, D), lambda i: (i, 0)),
        out_shape=jax.ShapeDtypeStruct((B, D), jnp.float32),
    )(h, W, b.reshape(1, D))


def kernel(x, emb, W, b):
    pooled = _pool(x, emb)
    return _dense(pooled, W, b)


# speedup vs baseline: 3.1636x; 1.0016x over previous
import functools

import jax
import jax.numpy as jnp
from jax import lax
from jax.experimental import pallas as pl
from jax.experimental.pallas import tpu as pltpu
from jax.experimental.pallas import tpu_sc as plsc

VOCAB = 1000000
D = 64
B = 16384
H = 200

_NC = 2
_NS = 16
_NW = _NC * _NS
_SPW = B // _NW
_S = 2
_CHUNK = 100
_NCH = _S * H // _CHUNK
_GROUPS = _SPW // _S
_NBUF = 2
_NIB = 4


def _pool_body(x2_hbm, emb_hbm, out_hbm, idx_v, rows_v, out_v,
               sr0, sr1, si0, si1, si2, si3):
    wid = lax.axis_index("s") * _NC + lax.axis_index("c")
    base = wid * _SPW
    sems_r = (sr0, sr1)
    sems_i = (si0, si1, si2, si3)

    def idx_row0(g):
        return (base + g * _S) * (H // _CHUNK)

    def idx_fetch(g, ib):
        pltpu.async_copy(
            x2_hbm.at[pl.ds(idx_row0(g), _NCH)], idx_v.at[ib], sems_i[ib]
        )

    def wait_idx(ib):
        pltpu.make_async_copy(
            x2_hbm.at[pl.ds(0, _NCH)], idx_v.at[ib], sems_i[ib]
        ).wait()

    def fire(ib, rb):
        for j in range(_NCH):
            pltpu.async_copy(
                emb_hbm.at[idx_v.at[ib, j]],
                rows_v.at[rb, pl.ds(j * _CHUNK, _CHUNK)],
                sems_r[rb],
            )

    def wait_rows(rb):
        pltpu.make_async_copy(
            emb_hbm.at[pl.ds(0, _S * H)], rows_v.at[rb], sems_r[rb]
        ).wait()

    def reduce(g, rb):
        for s in range(_S):
            r0 = s * H

            def red(r, acc):
                return tuple(
                    acc[k] + rows_v[rb, r0 + r, pl.ds(16 * k, 16)]
                    for k in range(4)
                )

            acc = lax.fori_loop(
                0, H, red,
                tuple(jnp.zeros((16,), jnp.float32) for _ in range(4)),
                unroll=8,
            )
            for k in range(4):
                out_v[g * _S + s, pl.ds(16 * k, 16)] = acc[k]

    pltpu.sync_copy(x2_hbm.at[pl.ds(idx_row0(0), _NCH)], idx_v.at[0])
    fire(0, 0)
    for p in range(1, _NIB):
        idx_fetch(p, p)

    def quad(qq, carry):
        for b in range(_NIB):
            g = _NIB * qq + b
            rb = b % _NBUF
            nrb = (b + 1) % _NBUF

            @pl.when(g + 1 < _GROUPS)
            def _():
                wait_idx((b + 1) % _NIB)
                fire((b + 1) % _NIB, nrb)

            wait_rows(rb)

            @pl.when(g + _NIB < _GROUPS)
            def _():
                idx_fetch(g + _NIB, b)

            reduce(g, rb)
        return carry

    lax.fori_loop(0, _GROUPS // _NIB, quad, 0)
    pltpu.sync_copy(out_v, out_hbm.at[pl.ds(base, _SPW)])


def _pool(x, emb):
    x2 = x.reshape(B * H // _CHUNK, _CHUNK)
    mesh = plsc.VectorSubcoreMesh(core_axis_name="c", subcore_axis_name="s")
    fn = functools.partial(
        pl.kernel,
        mesh=mesh,
        compiler_params=pltpu.CompilerParams(use_tc_tiling_on_sc=False),
        out_type=jax.ShapeDtypeStruct((B, D), jnp.float32),
        scratch_types=[
            pltpu.VMEM((_NIB, _NCH, _CHUNK), jnp.int32),
            pltpu.VMEM((_NBUF, _S * H, D), jnp.float32),
            pltpu.VMEM((_SPW, D), jnp.float32),
        ] + [pltpu.SemaphoreType.DMA] * (_NBUF + _NIB),
    )(_pool_body)
    return fn(x2, emb)


def _dense_body(h_ref, w_ref, b_ref, o_ref):
    o_ref[...] = jnp.tanh(
        jnp.dot(h_ref[...], w_ref[...], preferred_element_type=jnp.float32)
        + b_ref[...]
    )


def _dense(h, W, b):
    blk = 2048
    return pl.pallas_call(
        _dense_body,
        grid=(B // blk,),
        in_specs=[
            pl.BlockSpec((blk, D), lambda i: (i, 0)),
            pl.BlockSpec((D, D), lambda i: (0, 0)),
            pl.BlockSpec((1, D), lambda i: (0, 0)),
        ],
        out_specs=pl.BlockSpec((blk, D), lambda i: (i, 0)),
        out_shape=jax.ShapeDtypeStruct((B, D), jnp.float32),
    )(h, W, b.reshape(1, D))


def kernel(x, emb, W, b):
    pooled = _pool(x, emb)
    return _dense(pooled, W, b)
